# baseline (speedup 1.0000x reference)
import functools

import jax
import jax.numpy as jnp
from jax import lax
from jax.experimental import pallas as pl
from jax.experimental.pallas import tpu as pltpu
from jax.experimental.pallas import tpu_sc as plsc

_B, _S, _D, _F, _E = 2, 4096, 768, 1024, 16
_N = _B * _S
_THRESH = 0.8


_TG = 512


def _gate_body(x_ref, w_ref, b_ref, o_ref):
  o_ref[...] = (
      jnp.dot(x_ref[...], w_ref[...], preferred_element_type=jnp.float32)
      + b_ref[...]
  )


def _gate(x_flat, gate_w, gate_b):
  return pl.pallas_call(
      _gate_body,
      grid=(_N // _TG,),
      in_specs=[
          pl.BlockSpec((_TG, _D), lambda i: (i, 0)),
          pl.BlockSpec((_D, _E), lambda i: (0, 0)),
          pl.BlockSpec((1, _E), lambda i: (0, 0)),
      ],
      out_specs=pl.BlockSpec((_TG, _E), lambda i: (i, 0)),
      out_shape=jax.ShapeDtypeStruct((_N, _E), jnp.float32),
  )(x_flat, gate_w, gate_b.reshape(1, _E))


try:
  _INFO = plsc.get_sparse_core_info()
  _NC, _NS, _L = _INFO.num_cores, _INFO.num_subcores, _INFO.num_lanes
except ValueError:
  _NC, _NS, _L = 2, 16, 16
_NW = _NC * _NS
_TPW = _N // _NW


def _router_body(logits_hbm, rw_hbm, probs_hbm, ac_hbm, aa_hbm, log_v, rw_v,
                 p_v, ac_v, av_v):
  c = lax.axis_index("c")
  s = lax.axis_index("s")
  wid = s * _NC + c
  base = wid * _TPW
  pltpu.sync_copy(logits_hbm.at[pl.ds(base, _TPW), :], log_v)
  eidx = lax.iota(jnp.int32, _L)
  av_v[...] = jnp.zeros((_L,), jnp.float32)

  @plsc.parallel_loop(0, _TPW // _L, unroll=2)
  def group(g):
    acc = jnp.zeros((_L,), jnp.int32)
    for j in range(_L):
      i = g * _L + j
      lv = log_v[i, :]
      ex = jnp.exp(lv)
      z = jnp.sum(ex)
      p_v[i, :] = ex / z
      es, order = plsc.sort_key_val(ex, eidx, descending=True)
      shifted = plsc.cumsum(es) - es
      act = shifted < _THRESH * z
      ap = jnp.where(act, es, jnp.zeros_like(es))
      aw = ap / (jnp.sum(ap) + 1e-6 * z)
      plsc.store_scatter(rw_v.at[i], [order], aw)
      plsc.addupdate_scatter(av_v, [order], aw)
      acc = jnp.where(eidx == j, plsc.all_reduce_population_count(act), acc)
    ac_v[pl.ds(g * _L, _L)] = acc
  pltpu.sync_copy(rw_v, rw_hbm.at[pl.ds(base, _TPW), :])
  pltpu.sync_copy(p_v, probs_hbm.at[pl.ds(base, _TPW), :])
  pltpu.sync_copy(ac_v, ac_hbm.at[pl.ds(base, _TPW)])
  pltpu.sync_copy(av_v, aa_hbm.at[wid, :])


def _router(logits):
  f32 = jnp.float32
  return pl.kernel(
      _router_body,
      out_type=(
          jax.ShapeDtypeStruct((_N, _E), f32),
          jax.ShapeDtypeStruct((_N, _E), f32),
          jax.ShapeDtypeStruct((_N,), jnp.int32),
          jax.ShapeDtypeStruct((_NW, _E), f32),
      ),
      mesh=plsc.VectorSubcoreMesh(
          core_axis_name="c", subcore_axis_name="s"
      ),
      compiler_params=pltpu.CompilerParams(needs_layout_passes=False),
      scratch_types=[
          pltpu.VMEM((_TPW, _E), f32),
          pltpu.VMEM((_TPW, _E), f32),
          pltpu.VMEM((_TPW, _E), f32),
          pltpu.VMEM((_TPW,), jnp.int32),
          pltpu.VMEM((_L,), f32),
      ],
  )(logits)


_TT = 512


def _mega_body(aa_ref, rw_ref, p_ref, x_ref, w1_hbm, w3_hbm, w2_hbm,
               o_ref, lb_ref, le_ref,
               w1_v, w3_v, w2_v, sems, first_sm, ent_sm, tpe_v, psum_v):
  i = pl.program_id(0)
  nsteps = pl.num_programs(0)

  @pl.when(i == 0)
  def _prologue():
    anyv = jnp.max(aa_ref[...], axis=0)
    cand = jnp.where(anyv > 0.0, lax.iota(jnp.int32, _E), _E)
    fm = jnp.min(cand)
    first = jnp.where(fm == _E, 0, fm)
    first_sm[0] = first
    ent_sm[0] = 0.0
    tpe_v[...] = jnp.zeros((1, _E), jnp.float32)
    psum_v[...] = jnp.zeros((1, _E), jnp.float32)
    pltpu.make_async_copy(w1_hbm.at[first], w1_v, sems.at[0]).start()
    pltpu.make_async_copy(w3_hbm.at[first], w3_v, sems.at[1]).start()
    pltpu.make_async_copy(w2_hbm.at[first], w2_v, sems.at[2]).start()
    pltpu.make_async_copy(w1_hbm.at[first], w1_v, sems.at[0]).wait()
    pltpu.make_async_copy(w3_hbm.at[first], w3_v, sems.at[1]).wait()
    pltpu.make_async_copy(w2_hbm.at[first], w2_v, sems.at[2]).wait()

  rw = rw_ref[...]
  p = p_ref[...]
  mask = (rw > 0.0).astype(jnp.float32)
  tpe_v[...] += jnp.sum(mask, axis=0, keepdims=True)
  psum_v[...] += jnp.sum(p, axis=0, keepdims=True)
  ent_sm[0] += jnp.sum(p * jnp.log(p + 1e-6))

  xb = x_ref[...]
  h1 = jnp.dot(xb, w1_v[...], preferred_element_type=jnp.float32)
  h3 = jnp.dot(xb, w3_v[...], preferred_element_type=jnp.float32)
  h = h1 * jax.nn.sigmoid(h1) * h3
  out = jnp.dot(h, w2_v[...], preferred_element_type=jnp.float32)
  lane = lax.broadcasted_iota(jnp.int32, (_TT, _E), 1)
  scale = jnp.sum(
      jnp.where(lane == first_sm[0], rw, 0.0), axis=1, keepdims=True)
  o_ref[...] = out * scale

  @pl.when(i == nsteps - 1)
  def _epilogue():
    lb = _E * jnp.sum((tpe_v[0, :] / _N) * (psum_v[0, :] / _N))
    lb_ref[...] = jnp.full((1, 1), lb, jnp.float32)
    le_ref[...] = jnp.full((1, 1), -ent_sm[0] / _N, jnp.float32)


def _mega(aa, rw, probs, x_flat, w1, w3, w2):
  return pl.pallas_call(
      _mega_body,
      grid=(_N // _TT,),
      in_specs=[
          pl.BlockSpec((_NW, _E), lambda i: (0, 0)),
          pl.BlockSpec((_TT, _E), lambda i: (i, 0)),
          pl.BlockSpec((_TT, _E), lambda i: (i, 0)),
          pl.BlockSpec((_TT, _D), lambda i: (i, 0)),
          pl.BlockSpec(memory_space=pl.ANY),
          pl.BlockSpec(memory_space=pl.ANY),
          pl.BlockSpec(memory_space=pl.ANY),
      ],
      out_specs=[
          pl.BlockSpec((_TT, _D), lambda i: (i, 0)),
          pl.BlockSpec((1, 1), lambda i: (0, 0)),
          pl.BlockSpec((1, 1), lambda i: (0, 0)),
      ],
      out_shape=[
          jax.ShapeDtypeStruct((_N, _D), jnp.float32),
          jax.ShapeDtypeStruct((1, 1), jnp.float32),
          jax.ShapeDtypeStruct((1, 1), jnp.float32),
      ],
      scratch_shapes=[
          pltpu.VMEM((_D, _F), jnp.float32),
          pltpu.VMEM((_D, _F), jnp.float32),
          pltpu.VMEM((_F, _D), jnp.float32),
          pltpu.SemaphoreType.DMA((3,)),
          pltpu.SMEM((1,), jnp.int32),
          pltpu.SMEM((1,), jnp.float32),
          pltpu.VMEM((1, _E), jnp.float32),
          pltpu.VMEM((1, _E), jnp.float32),
      ],
  )(aa, rw, probs, x_flat, w1, w3, w2)


@jax.jit
def kernel(x, gate_w, gate_b, w1, w3, w2):
  x_flat = x.reshape(_N, _D)
  logits = _gate(x_flat, gate_w, gate_b)
  rw, probs, ac, aa = _router(logits)
  out, lb, le = _mega(aa, rw, probs, x_flat, w1, w3, w2)
  return (
      out.reshape(_B, _S, _D),
      lb.reshape(()),
      le.reshape(()),
      ac.reshape(_B, _S),
  )


# speedup vs baseline: 1.0153x; 1.0153x over previous
import functools

import jax
import jax.numpy as jnp
from jax import lax
from jax.experimental import pallas as pl
from jax.experimental.pallas import tpu as pltpu
from jax.experimental.pallas import tpu_sc as plsc

_B, _S, _D, _F, _E = 2, 4096, 768, 1024, 16
_N = _B * _S
_THRESH = 0.8


_TG = 512


def _gate_body(x_ref, w_ref, b_ref, o_ref):
  o_ref[...] = (
      jnp.dot(x_ref[...], w_ref[...], preferred_element_type=jnp.float32)
      + b_ref[...]
  )


def _gate(x_flat, gate_w, gate_b):
  return pl.pallas_call(
      _gate_body,
      grid=(_N // _TG,),
      in_specs=[
          pl.BlockSpec((_TG, _D), lambda i: (i, 0)),
          pl.BlockSpec((_D, _E), lambda i: (0, 0)),
          pl.BlockSpec((1, _E), lambda i: (0, 0)),
      ],
      out_specs=pl.BlockSpec((_TG, _E), lambda i: (i, 0)),
      out_shape=jax.ShapeDtypeStruct((_N, _E), jnp.float32),
  )(x_flat, gate_w, gate_b.reshape(1, _E))


try:
  _INFO = plsc.get_sparse_core_info()
  _NC, _NS, _L = _INFO.num_cores, _INFO.num_subcores, _INFO.num_lanes
except ValueError:
  _NC, _NS, _L = 2, 16, 16
_NW = _NC * _NS
_TPW = _N // _NW


def _router_body(logits_hbm, rw_hbm, probs_hbm, ac_hbm, aa_hbm, log_v, rw_v,
                 p_v, ac_v, av_v):
  c = lax.axis_index("c")
  s = lax.axis_index("s")
  wid = s * _NC + c
  base = wid * _TPW
  pltpu.sync_copy(logits_hbm.at[pl.ds(base, _TPW), :], log_v)
  eidx = lax.iota(jnp.int32, _L)
  av_v[...] = jnp.zeros((_L,), jnp.float32)

  @plsc.parallel_loop(0, _TPW // _L, unroll=2)
  def group(g):
    acc = jnp.zeros((_L,), jnp.int32)
    for j in range(_L):
      i = g * _L + j
      lv = log_v[i, :]
      ex = jnp.exp(lv)
      z = jnp.sum(ex)
      p_v[i, :] = ex / z
      es, order = plsc.sort_key_val(ex, eidx, descending=True)
      shifted = plsc.cumsum(es) - es
      act = shifted < _THRESH * z
      ap = jnp.where(act, es, jnp.zeros_like(es))
      aw = ap / (jnp.sum(ap) + 1e-6 * z)
      plsc.store_scatter(rw_v.at[i], [order], aw)
      plsc.addupdate_scatter(av_v, [order], aw)
      acc = jnp.where(eidx == j, plsc.all_reduce_population_count(act), acc)
    ac_v[pl.ds(g * _L, _L)] = acc
  pltpu.sync_copy(rw_v, rw_hbm.at[pl.ds(base, _TPW), :])
  pltpu.sync_copy(p_v, probs_hbm.at[pl.ds(base, _TPW), :])
  pltpu.sync_copy(ac_v, ac_hbm.at[pl.ds(base, _TPW)])
  pltpu.sync_copy(av_v, aa_hbm.at[wid, :])


def _router(logits):
  f32 = jnp.float32
  return pl.kernel(
      _router_body,
      out_type=(
          jax.ShapeDtypeStruct((_N, _E), f32),
          jax.ShapeDtypeStruct((_N, _E), f32),
          jax.ShapeDtypeStruct((_N,), jnp.int32),
          jax.ShapeDtypeStruct((_NW, _E), f32),
      ),
      mesh=plsc.VectorSubcoreMesh(
          core_axis_name="c", subcore_axis_name="s"
      ),
      compiler_params=pltpu.CompilerParams(needs_layout_passes=False),
      scratch_types=[
          pltpu.VMEM((_TPW, _E), f32),
          pltpu.VMEM((_TPW, _E), f32),
          pltpu.VMEM((_TPW, _E), f32),
          pltpu.VMEM((_TPW,), jnp.int32),
          pltpu.VMEM((_L,), f32),
      ],
  )(logits)


_TT = 1024


def _mega_body(aa_ref, rw_ref, p_ref, x_ref, w1_hbm, w3_hbm, w2_hbm,
               o_ref, lb_ref, le_ref,
               w1_v, w3_v, w2_v, sems, first_sm, ent_sm, tpe_v, psum_v):
  i = pl.program_id(0)
  nsteps = pl.num_programs(0)

  @pl.when(i == 0)
  def _prologue():
    anyv = jnp.max(aa_ref[...], axis=0)
    cand = jnp.where(anyv > 0.0, lax.iota(jnp.int32, _E), _E)
    fm = jnp.min(cand)
    first = jnp.where(fm == _E, 0, fm)
    first_sm[0] = first
    ent_sm[0] = 0.0
    tpe_v[...] = jnp.zeros((1, _E), jnp.float32)
    psum_v[...] = jnp.zeros((1, _E), jnp.float32)
    pltpu.make_async_copy(w1_hbm.at[first], w1_v, sems.at[0]).start()
    pltpu.make_async_copy(w3_hbm.at[first], w3_v, sems.at[1]).start()
    pltpu.make_async_copy(w2_hbm.at[first], w2_v, sems.at[2]).start()
    pltpu.make_async_copy(w1_hbm.at[first], w1_v, sems.at[0]).wait()
    pltpu.make_async_copy(w3_hbm.at[first], w3_v, sems.at[1]).wait()
    pltpu.make_async_copy(w2_hbm.at[first], w2_v, sems.at[2]).wait()

  rw = rw_ref[...]
  p = p_ref[...]
  mask = (rw > 0.0).astype(jnp.float32)
  tpe_v[...] += jnp.sum(mask, axis=0, keepdims=True)
  psum_v[...] += jnp.sum(p, axis=0, keepdims=True)
  ent_sm[0] += jnp.sum(p * jnp.log(p + 1e-6))

  xb = x_ref[...]
  h1 = jnp.dot(xb, w1_v[...], preferred_element_type=jnp.float32)
  h3 = jnp.dot(xb, w3_v[...], preferred_element_type=jnp.float32)
  h = h1 * jax.nn.sigmoid(h1) * h3
  out = jnp.dot(h, w2_v[...], preferred_element_type=jnp.float32)
  lane = lax.broadcasted_iota(jnp.int32, (_TT, _E), 1)
  scale = jnp.sum(
      jnp.where(lane == first_sm[0], rw, 0.0), axis=1, keepdims=True)
  o_ref[...] = out * scale

  @pl.when(i == nsteps - 1)
  def _epilogue():
    lb = _E * jnp.sum((tpe_v[0, :] / _N) * (psum_v[0, :] / _N))
    lb_ref[...] = jnp.full((1, 1), lb, jnp.float32)
    le_ref[...] = jnp.full((1, 1), -ent_sm[0] / _N, jnp.float32)


def _mega(aa, rw, probs, x_flat, w1, w3, w2):
  return pl.pallas_call(
      _mega_body,
      grid=(_N // _TT,),
      in_specs=[
          pl.BlockSpec((_NW, _E), lambda i: (0, 0)),
          pl.BlockSpec((_TT, _E), lambda i: (i, 0)),
          pl.BlockSpec((_TT, _E), lambda i: (i, 0)),
          pl.BlockSpec((_TT, _D), lambda i: (i, 0)),
          pl.BlockSpec(memory_space=pl.ANY),
          pl.BlockSpec(memory_space=pl.ANY),
          pl.BlockSpec(memory_space=pl.ANY),
      ],
      out_specs=[
          pl.BlockSpec((_TT, _D), lambda i: (i, 0)),
          pl.BlockSpec((1, 1), lambda i: (0, 0)),
          pl.BlockSpec((1, 1), lambda i: (0, 0)),
      ],
      out_shape=[
          jax.ShapeDtypeStruct((_N, _D), jnp.float32),
          jax.ShapeDtypeStruct((1, 1), jnp.float32),
          jax.ShapeDtypeStruct((1, 1), jnp.float32),
      ],
      scratch_shapes=[
          pltpu.VMEM((_D, _F), jnp.float32),
          pltpu.VMEM((_D, _F), jnp.float32),
          pltpu.VMEM((_F, _D), jnp.float32),
          pltpu.SemaphoreType.DMA((3,)),
          pltpu.SMEM((1,), jnp.int32),
          pltpu.SMEM((1,), jnp.float32),
          pltpu.VMEM((1, _E), jnp.float32),
          pltpu.VMEM((1, _E), jnp.float32),
      ],
  )(aa, rw, probs, x_flat, w1, w3, w2)


@jax.jit
def kernel(x, gate_w, gate_b, w1, w3, w2):
  x_flat = x.reshape(_N, _D)
  logits = _gate(x_flat, gate_w, gate_b)
  rw, probs, ac, aa = _router(logits)
  out, lb, le = _mega(aa, rw, probs, x_flat, w1, w3, w2)
  return (
      out.reshape(_B, _S, _D),
      lb.reshape(()),
      le.reshape(()),
      ac.reshape(_B, _S),
  )


# speedup vs baseline: 1.0434x; 1.0277x over previous
import functools

import jax
import jax.numpy as jnp
from jax import lax
from jax.experimental import pallas as pl
from jax.experimental.pallas import tpu as pltpu
from jax.experimental.pallas import tpu_sc as plsc

_B, _S, _D, _F, _E = 2, 4096, 768, 1024, 16
_N = _B * _S
_THRESH = 0.8


_TG = 1024


def _gate_body(x_ref, w_ref, b_ref, o_ref):
  o_ref[...] = (
      jnp.dot(x_ref[...], w_ref[...], preferred_element_type=jnp.float32)
      + b_ref[...]
  )


def _gate(x_flat, gate_w, gate_b):
  return pl.pallas_call(
      _gate_body,
      grid=(_N // _TG,),
      in_specs=[
          pl.BlockSpec((_TG, _D), lambda i: (i, 0)),
          pl.BlockSpec((_D, _E), lambda i: (0, 0)),
          pl.BlockSpec((1, _E), lambda i: (0, 0)),
      ],
      out_specs=pl.BlockSpec((_TG, _E), lambda i: (i, 0)),
      out_shape=jax.ShapeDtypeStruct((_N, _E), jnp.float32),
  )(x_flat, gate_w, gate_b.reshape(1, _E))


try:
  _INFO = plsc.get_sparse_core_info()
  _NC, _NS, _L = _INFO.num_cores, _INFO.num_subcores, _INFO.num_lanes
except ValueError:
  _NC, _NS, _L = 2, 16, 16
_NW = _NC * _NS
_TPW = _N // _NW


def _router_body(logits_hbm, rw_hbm, probs_hbm, ac_hbm, aa_hbm, log_v, rw_v,
                 p_v, ac_v, av_v):
  c = lax.axis_index("c")
  s = lax.axis_index("s")
  wid = s * _NC + c
  base = wid * _TPW
  pltpu.sync_copy(logits_hbm.at[pl.ds(base, _TPW), :], log_v)
  eidx = lax.iota(jnp.int32, _L)
  av_v[...] = jnp.zeros((_L,), jnp.float32)

  @plsc.parallel_loop(0, _TPW // _L, unroll=4)
  def group(g):
    acc = jnp.zeros((_L,), jnp.int32)
    for j in range(_L):
      i = g * _L + j
      lv = log_v[i, :]
      ex = jnp.exp(lv)
      z = jnp.sum(ex)
      p_v[i, :] = ex / z
      es, order = plsc.sort_key_val(ex, eidx, descending=True)
      shifted = plsc.cumsum(es) - es
      act = shifted < _THRESH * z
      ap = jnp.where(act, es, jnp.zeros_like(es))
      aw = ap / (jnp.sum(ap) + 1e-6 * z)
      plsc.store_scatter(rw_v.at[i], [order], aw)
      plsc.addupdate_scatter(av_v, [order], aw)
      acc = jnp.where(eidx == j, plsc.all_reduce_population_count(act), acc)
    ac_v[pl.ds(g * _L, _L)] = acc
  pltpu.sync_copy(rw_v, rw_hbm.at[pl.ds(base, _TPW), :])
  pltpu.sync_copy(p_v, probs_hbm.at[pl.ds(base, _TPW), :])
  pltpu.sync_copy(ac_v, ac_hbm.at[pl.ds(base, _TPW)])
  pltpu.sync_copy(av_v, aa_hbm.at[wid, :])


def _router(logits):
  f32 = jnp.float32
  return pl.kernel(
      _router_body,
      out_type=(
          jax.ShapeDtypeStruct((_N, _E), f32),
          jax.ShapeDtypeStruct((_N, _E), f32),
          jax.ShapeDtypeStruct((_N,), jnp.int32),
          jax.ShapeDtypeStruct((_NW, _E), f32),
      ),
      mesh=plsc.VectorSubcoreMesh(
          core_axis_name="c", subcore_axis_name="s"
      ),
      compiler_params=pltpu.CompilerParams(needs_layout_passes=False),
      scratch_types=[
          pltpu.VMEM((_TPW, _E), f32),
          pltpu.VMEM((_TPW, _E), f32),
          pltpu.VMEM((_TPW, _E), f32),
          pltpu.VMEM((_TPW,), jnp.int32),
          pltpu.VMEM((_L,), f32),
      ],
  )(logits)


_TT = 1024


def _mega_body(aa_ref, rw_ref, p_ref, x_ref, w1_hbm, w3_hbm, w2_hbm,
               o_ref, lb_ref, le_ref,
               w1_v, w3_v, w2_v, sems, first_sm, ent_sm, tpe_v, psum_v):
  i = pl.program_id(0)
  nsteps = pl.num_programs(0)

  @pl.when(i == 0)
  def _prologue():
    anyv = jnp.max(aa_ref[...], axis=0)
    cand = jnp.where(anyv > 0.0, lax.iota(jnp.int32, _E), _E)
    fm = jnp.min(cand)
    first = jnp.where(fm == _E, 0, fm)
    first_sm[0] = first
    ent_sm[0] = 0.0
    tpe_v[...] = jnp.zeros((1, _E), jnp.float32)
    psum_v[...] = jnp.zeros((1, _E), jnp.float32)
    pltpu.make_async_copy(w1_hbm.at[first], w1_v, sems.at[0]).start()
    pltpu.make_async_copy(w3_hbm.at[first], w3_v, sems.at[1]).start()
    pltpu.make_async_copy(w2_hbm.at[first], w2_v, sems.at[2]).start()
    pltpu.make_async_copy(w1_hbm.at[first], w1_v, sems.at[0]).wait()
    pltpu.make_async_copy(w3_hbm.at[first], w3_v, sems.at[1]).wait()
    pltpu.make_async_copy(w2_hbm.at[first], w2_v, sems.at[2]).wait()

  rw = rw_ref[...]
  p = p_ref[...]
  mask = (rw > 0.0).astype(jnp.float32)
  tpe_v[...] += jnp.sum(mask, axis=0, keepdims=True)
  psum_v[...] += jnp.sum(p, axis=0, keepdims=True)
  ent_sm[0] += jnp.sum(p * jnp.log(p + 1e-6))

  xb = x_ref[...]
  h1 = jnp.dot(xb, w1_v[...], preferred_element_type=jnp.float32)
  h3 = jnp.dot(xb, w3_v[...], preferred_element_type=jnp.float32)
  h = h1 * jax.nn.sigmoid(h1) * h3
  out = jnp.dot(h, w2_v[...], preferred_element_type=jnp.float32)
  lane = lax.broadcasted_iota(jnp.int32, (_TT, _E), 1)
  scale = jnp.sum(
      jnp.where(lane == first_sm[0], rw, 0.0), axis=1, keepdims=True)
  o_ref[...] = out * scale

  @pl.when(i == nsteps - 1)
  def _epilogue():
    lb = _E * jnp.sum((tpe_v[0, :] / _N) * (psum_v[0, :] / _N))
    lb_ref[...] = jnp.full((1, 1), lb, jnp.float32)
    le_ref[...] = jnp.full((1, 1), -ent_sm[0] / _N, jnp.float32)


def _mega(aa, rw, probs, x_flat, w1, w3, w2):
  return pl.pallas_call(
      _mega_body,
      grid=(_N // _TT,),
      in_specs=[
          pl.BlockSpec((_NW, _E), lambda i: (0, 0)),
          pl.BlockSpec((_TT, _E), lambda i: (i, 0)),
          pl.BlockSpec((_TT, _E), lambda i: (i, 0)),
          pl.BlockSpec((_TT, _D), lambda i: (i, 0)),
          pl.BlockSpec(memory_space=pl.ANY),
          pl.BlockSpec(memory_space=pl.ANY),
          pl.BlockSpec(memory_space=pl.ANY),
      ],
      out_specs=[
          pl.BlockSpec((_TT, _D), lambda i: (i, 0)),
          pl.BlockSpec((1, 1), lambda i: (0, 0)),
          pl.BlockSpec((1, 1), lambda i: (0, 0)),
      ],
      out_shape=[
          jax.ShapeDtypeStruct((_N, _D), jnp.float32),
          jax.ShapeDtypeStruct((1, 1), jnp.float32),
          jax.ShapeDtypeStruct((1, 1), jnp.float32),
      ],
      scratch_shapes=[
          pltpu.VMEM((_D, _F), jnp.float32),
          pltpu.VMEM((_D, _F), jnp.float32),
          pltpu.VMEM((_F, _D), jnp.float32),
          pltpu.SemaphoreType.DMA((3,)),
          pltpu.SMEM((1,), jnp.int32),
          pltpu.SMEM((1,), jnp.float32),
          pltpu.VMEM((1, _E), jnp.float32),
          pltpu.VMEM((1, _E), jnp.float32),
      ],
  )(aa, rw, probs, x_flat, w1, w3, w2)


@jax.jit
def kernel(x, gate_w, gate_b, w1, w3, w2):
  x_flat = x.reshape(_N, _D)
  logits = _gate(x_flat, gate_w, gate_b)
  rw, probs, ac, aa = _router(logits)
  out, lb, le = _mega(aa, rw, probs, x_flat, w1, w3, w2)
  return (
      out.reshape(_B, _S, _D),
      lb.reshape(()),
      le.reshape(()),
      ac.reshape(_B, _S),
  )
